# Initial kernel scaffold; baseline (speedup 1.0000x reference)
import jax
import jax.numpy as jnp
from jax.experimental import pallas as pl


def kernel(features, labels):
    raise NotImplementedError("write your pallas kernel here")


# speedup vs baseline: 1.2766x; 1.2766x over previous
import functools

import jax
import jax.numpy as jnp
from jax import lax
from jax.experimental import pallas as pl
from jax.experimental.pallas import tpu as pltpu
from jax.experimental.pallas import tpu_sc as plsc

N = 160000
D = 256
C = 1000
CP = 1024
CHUNK = 128
NCHUNKS = N // CHUNK
NCORES = 2
NSUB = 16
NW = NCORES * NSUB

BR = 2000
NB = N // BR


def _centers_body(feat_ref, lab_ref, cen_ref, cnt_ref, sums_acc, cnt_acc):
    i = pl.program_id(0)

    @pl.when(i == 0)
    def _():
        sums_acc[...] = jnp.zeros((CP, D), jnp.float32)
        cnt_acc[...] = jnp.zeros((CP, 128), jnp.float32)

    lab = lab_ref[0, 0, :]
    onehot = (lab[:, None] ==
              lax.broadcasted_iota(jnp.int32, (BR, CP), 1)).astype(jnp.float32)
    sums_acc[...] += lax.dot_general(
        onehot, feat_ref[...], (((0,), (0,)), ((), ())),
        preferred_element_type=jnp.float32)
    cnt_acc[...] += jnp.sum(onehot, axis=0)[:, None]

    @pl.when(i == NB - 1)
    def _():
        cnt = cnt_acc[...][:, 0:1]
        safe = jnp.where(cnt > 0, cnt, 1.0)
        cen_ref[...] = jnp.where(cnt > 0, sums_acc[...] / safe, 0.0)
        cnt_ref[...] = cnt_acc[...]


def _compute_centers(features, lab3d):
    return pl.pallas_call(
        _centers_body,
        grid=(NB,),
        in_specs=[
            pl.BlockSpec((BR, D), lambda i: (i, 0)),
            pl.BlockSpec((1, 1, BR), lambda i: (i, 0, 0)),
        ],
        out_specs=[
            pl.BlockSpec((CP, D), lambda i: (0, 0)),
            pl.BlockSpec((CP, 128), lambda i: (0, 0)),
        ],
        out_shape=[
            jax.ShapeDtypeStruct((CP, D), jnp.float32),
            jax.ShapeDtypeStruct((CP, 128), jnp.float32),
        ],
        scratch_shapes=[
            pltpu.VMEM((CP, D), jnp.float32),
            pltpu.VMEM((CP, 128), jnp.float32),
        ],
    )(features, lab3d)


def _gather_body(centers, lab, out, rows_v, idx_v, sem):
    cid = lax.axis_index("c")
    sid = lax.axis_index("s")
    wid = sid * NCORES + cid
    base = (wid * NCHUNKS) // NW
    end = ((wid + 1) * NCHUNKS) // NW

    def body(j, carry):
        start = j * CHUNK
        pltpu.sync_copy(lab.at[pl.ds(start, CHUNK)], idx_v)
        pltpu.async_copy(centers.at[idx_v], rows_v, sem).wait()
        pltpu.sync_copy(rows_v, out.at[pl.ds(start, CHUNK)])
        return carry

    lax.fori_loop(base, end, body, 0)


@functools.lru_cache(maxsize=None)
def _gather_kernel():
    mesh = plsc.VectorSubcoreMesh(core_axis_name="c", subcore_axis_name="s")
    return pl.kernel(
        _gather_body,
        out_type=jax.ShapeDtypeStruct((N, D), jnp.float32),
        mesh=mesh,
        scratch_types=[
            pltpu.VMEM((CHUNK, D), jnp.float32),
            pltpu.VMEM((CHUNK,), jnp.int32),
            pltpu.SemaphoreType.DMA,
        ],
    )


def _dist_body(feat_ref, gath_ref, lab_ref, cnt_ref, loss_ref, acc_ref):
    i = pl.program_id(0)

    @pl.when(i == 0)
    def _():
        acc_ref[...] = jnp.zeros((1, CP), jnp.float32)

    diff = feat_ref[...] - gath_ref[...] + 1e-6
    dist = jnp.sqrt(jnp.sum(diff * diff, axis=1))
    lab = lab_ref[0, 0, :]
    onehot = (lab[:, None] ==
              lax.broadcasted_iota(jnp.int32, (BR, CP), 1)).astype(jnp.float32)
    acc_ref[...] += jnp.sum(onehot * dist[:, None], axis=0)[None, :]

    @pl.when(i == NB - 1)
    def _():
        cnt = cnt_ref[...][:, 0]
        ds = acc_ref[0, :]
        safe = jnp.where(cnt > 0, cnt, 1.0)
        loss_ref[...] = jnp.sum(jnp.where(cnt > 0, ds / safe, 0.0))[None, None]


def _distance_loss(features, gathered, lab3d, cnt):
    return pl.pallas_call(
        _dist_body,
        grid=(NB,),
        in_specs=[
            pl.BlockSpec((BR, D), lambda i: (i, 0)),
            pl.BlockSpec((BR, D), lambda i: (i, 0)),
            pl.BlockSpec((1, 1, BR), lambda i: (i, 0, 0)),
            pl.BlockSpec((CP, 128), lambda i: (0, 0)),
        ],
        out_specs=pl.BlockSpec((1, 1), lambda i: (0, 0)),
        out_shape=jax.ShapeDtypeStruct((1, 1), jnp.float32),
        scratch_shapes=[pltpu.VMEM((1, CP), jnp.float32)],
    )(features, gathered, lab3d, cnt)


def kernel(features, labels):
    lab32 = labels.astype(jnp.int32)
    lab3d = lab32.reshape(NB, 1, BR)
    centers, cnt = _compute_centers(features, lab3d)
    gathered = _gather_kernel()(centers, lab32)
    loss = _distance_loss(features, gathered, lab3d, cnt)
    return loss[0, 0]


# speedup vs baseline: 1.5052x; 1.1790x over previous
import functools

import jax
import jax.numpy as jnp
from jax import lax
from jax.experimental import pallas as pl
from jax.experimental.pallas import tpu as pltpu
from jax.experimental.pallas import tpu_sc as plsc

N = 160000
D = 256
C = 1000
CP = 1024
CHUNK = 128
NCHUNKS = N // CHUNK
NCORES = 2
NSUB = 16
NW = NCORES * NSUB

BR = 2000
NB = N // BR


def _centers_body(feat_ref, lab_ref, cen_ref, cnt_ref, sums_acc, cnt_acc):
    i = pl.program_id(0)

    @pl.when(i == 0)
    def _():
        sums_acc[...] = jnp.zeros((CP, D), jnp.float32)
        cnt_acc[...] = jnp.zeros((CP, 128), jnp.float32)

    lab = lab_ref[0, 0, :]
    onehot = (lab[:, None] ==
              lax.broadcasted_iota(jnp.int32, (BR, CP), 1)).astype(jnp.float32)
    sums_acc[...] += lax.dot_general(
        onehot.astype(jnp.bfloat16), feat_ref[...].astype(jnp.bfloat16),
        (((0,), (0,)), ((), ())),
        preferred_element_type=jnp.float32)
    cnt_acc[...] += jnp.sum(onehot, axis=0)[:, None]

    @pl.when(i == NB - 1)
    def _():
        cnt = cnt_acc[...][:, 0:1]
        safe = jnp.where(cnt > 0, cnt, 1.0)
        cen_ref[...] = jnp.where(cnt > 0, sums_acc[...] / safe, 0.0)
        cnt_ref[...] = cnt_acc[...]


def _compute_centers(features, lab3d):
    return pl.pallas_call(
        _centers_body,
        grid=(NB,),
        in_specs=[
            pl.BlockSpec((BR, D), lambda i: (i, 0)),
            pl.BlockSpec((1, 1, BR), lambda i: (i, 0, 0)),
        ],
        out_specs=[
            pl.BlockSpec((CP, D), lambda i: (0, 0)),
            pl.BlockSpec((CP, 128), lambda i: (0, 0)),
        ],
        out_shape=[
            jax.ShapeDtypeStruct((CP, D), jnp.float32),
            jax.ShapeDtypeStruct((CP, 128), jnp.float32),
        ],
        scratch_shapes=[
            pltpu.VMEM((CP, D), jnp.float32),
            pltpu.VMEM((CP, 128), jnp.float32),
        ],
    )(features, lab3d)


WPW = N // NW
NFULL = WPW // CHUNK
NPAIR = NFULL // 2
TAIL = WPW - NFULL * CHUNK


def _gather_body(centers, lab, out, rows0, rows1, tail_v, idx_all, tidx_v,
                 sem0, sem1):
    cid = lax.axis_index("c")
    sid = lax.axis_index("s")
    wid = sid * NCORES + cid
    wbase = wid * WPW

    pltpu.sync_copy(lab.at[pl.ds(wbase, NFULL * CHUNK)], idx_all)
    pltpu.sync_copy(lab.at[pl.ds(wbase + NFULL * CHUNK, TAIL)], tidx_v)

    def pair(i, carry):
        s0 = i * 2 * CHUNK
        s1 = s0 + CHUNK
        g0 = pltpu.async_copy(centers.at[idx_all.at[pl.ds(s0, CHUNK)]],
                              rows0, sem0)
        g1 = pltpu.async_copy(centers.at[idx_all.at[pl.ds(s1, CHUNK)]],
                              rows1, sem1)
        g0.wait()
        pltpu.sync_copy(rows0, out.at[pl.ds(wbase + s0, CHUNK)])
        g1.wait()
        pltpu.sync_copy(rows1, out.at[pl.ds(wbase + s1, CHUNK)])
        return carry

    lax.fori_loop(0, NPAIR, pair, 0)

    s_last = NPAIR * 2 * CHUNK
    g0 = pltpu.async_copy(centers.at[idx_all.at[pl.ds(s_last, CHUNK)]],
                          rows0, sem0)
    g1 = pltpu.async_copy(centers.at[tidx_v], tail_v, sem1)
    g0.wait()
    pltpu.sync_copy(rows0, out.at[pl.ds(wbase + s_last, CHUNK)])
    g1.wait()
    pltpu.sync_copy(tail_v, out.at[pl.ds(wbase + s_last + CHUNK, TAIL)])


@functools.lru_cache(maxsize=None)
def _gather_kernel():
    mesh = plsc.VectorSubcoreMesh(core_axis_name="c", subcore_axis_name="s")
    return pl.kernel(
        _gather_body,
        out_type=jax.ShapeDtypeStruct((N, D), jnp.float32),
        mesh=mesh,
        scratch_types=[
            pltpu.VMEM((CHUNK, D), jnp.float32),
            pltpu.VMEM((CHUNK, D), jnp.float32),
            pltpu.VMEM((TAIL, D), jnp.float32),
            pltpu.VMEM((NFULL * CHUNK,), jnp.int32),
            pltpu.VMEM((TAIL,), jnp.int32),
            pltpu.SemaphoreType.DMA,
            pltpu.SemaphoreType.DMA,
        ],
    )


def _dist_body(feat_ref, gath_ref, lab_ref, cnt_ref, loss_ref, acc_ref):
    i = pl.program_id(0)

    @pl.when(i == 0)
    def _():
        acc_ref[...] = jnp.zeros((1, CP), jnp.float32)

    diff = feat_ref[...] - gath_ref[...] + 1e-6
    dist = jnp.sqrt(jnp.sum(diff * diff, axis=1))
    lab = lab_ref[0, 0, :]
    onehot = (lab[:, None] ==
              lax.broadcasted_iota(jnp.int32, (BR, CP), 1)).astype(jnp.float32)
    acc_ref[...] += lax.dot_general(
        dist[None, :], onehot, (((1,), (0,)), ((), ())),
        preferred_element_type=jnp.float32)

    @pl.when(i == NB - 1)
    def _():
        cnt = cnt_ref[...][:, 0]
        ds = acc_ref[0, :]
        safe = jnp.where(cnt > 0, cnt, 1.0)
        loss_ref[...] = jnp.sum(jnp.where(cnt > 0, ds / safe, 0.0))[None, None]


def _distance_loss(features, gathered, lab3d, cnt):
    return pl.pallas_call(
        _dist_body,
        grid=(NB,),
        in_specs=[
            pl.BlockSpec((BR, D), lambda i: (i, 0)),
            pl.BlockSpec((BR, D), lambda i: (i, 0)),
            pl.BlockSpec((1, 1, BR), lambda i: (i, 0, 0)),
            pl.BlockSpec((CP, 128), lambda i: (0, 0)),
        ],
        out_specs=pl.BlockSpec((1, 1), lambda i: (0, 0)),
        out_shape=jax.ShapeDtypeStruct((1, 1), jnp.float32),
        scratch_shapes=[pltpu.VMEM((1, CP), jnp.float32)],
    )(features, gathered, lab3d, cnt)


def kernel(features, labels):
    lab32 = labels.astype(jnp.int32)
    lab3d = lab32.reshape(NB, 1, BR)
    centers, cnt = _compute_centers(features, lab3d)
    gathered = _gather_kernel()(centers, lab32)
    loss = _distance_loss(features, gathered, lab3d, cnt)
    return loss[0, 0]


# speedup vs baseline: 2.0534x; 1.3642x over previous
import functools

import jax
import jax.numpy as jnp
from jax import lax
from jax.experimental import pallas as pl
from jax.experimental.pallas import tpu as pltpu
from jax.experimental.pallas import tpu_sc as plsc

N = 160000
D = 256
C = 1000
CP = 1024
CHUNK = 128
NCHUNKS = N // CHUNK
NCORES = 2
NSUB = 16
NW = NCORES * NSUB

BR = 2000
NB = N // BR


def _centers_body(feat_ref, lab_ref, cen_ref, cnt_ref, sums_acc, cnt_acc):
    i = pl.program_id(0)

    @pl.when(i == 0)
    def _():
        sums_acc[...] = jnp.zeros((CP, D), jnp.float32)
        cnt_acc[...] = jnp.zeros((CP, 128), jnp.float32)

    lab = lab_ref[0, 0, :]
    onehot = (lab[:, None] ==
              lax.broadcasted_iota(jnp.int32, (BR, CP), 1)).astype(jnp.float32)
    sums_acc[...] += lax.dot_general(
        onehot.astype(jnp.bfloat16), feat_ref[...].astype(jnp.bfloat16),
        (((0,), (0,)), ((), ())),
        preferred_element_type=jnp.float32)
    cnt_acc[...] += jnp.sum(onehot, axis=0)[:, None]

    @pl.when(i == NB - 1)
    def _():
        cnt = cnt_acc[...][:, 0:1]
        safe = jnp.where(cnt > 0, cnt, 1.0)
        cen = jnp.where(cnt > 0, sums_acc[...] / safe, 0.0)
        lo = lax.bitcast_convert_type(
            cen[:, :128].astype(jnp.bfloat16), jnp.uint16).astype(jnp.uint32)
        hi = lax.bitcast_convert_type(
            cen[:, 128:].astype(jnp.bfloat16), jnp.uint16).astype(jnp.uint32)
        cen_ref[...] = lax.bitcast_convert_type((hi << 16) | lo, jnp.float32)
        cnt_ref[...] = cnt_acc[...]


def _compute_centers(features, lab3d):
    return pl.pallas_call(
        _centers_body,
        grid=(NB,),
        in_specs=[
            pl.BlockSpec((BR, D), lambda i: (i, 0)),
            pl.BlockSpec((1, 1, BR), lambda i: (i, 0, 0)),
        ],
        out_specs=[
            pl.BlockSpec((CP, 128), lambda i: (0, 0)),
            pl.BlockSpec((CP, 128), lambda i: (0, 0)),
        ],
        out_shape=[
            jax.ShapeDtypeStruct((CP, 128), jnp.float32),
            jax.ShapeDtypeStruct((CP, 128), jnp.float32),
        ],
        scratch_shapes=[
            pltpu.VMEM((CP, D), jnp.float32),
            pltpu.VMEM((CP, 128), jnp.float32),
        ],
    )(features, lab3d)


WPW = N // NW
NFULL = WPW // CHUNK
TAIL = WPW - NFULL * CHUNK
NSLOT = 6
NROUND = NFULL // NSLOT
NLEFT = NFULL - NROUND * NSLOT


def _gather_body(centers, lab, out, *refs):
    bufs = refs[0:NSLOT]
    tail_v = refs[NSLOT]
    idx_all = refs[NSLOT + 1]
    tidx_v = refs[NSLOT + 2]
    gsems = refs[NSLOT + 3:NSLOT + 3 + NSLOT]
    wsems = refs[NSLOT * 2 + 3:NSLOT * 2 + 3 + NSLOT]
    tsem = refs[NSLOT * 3 + 3]

    cid = lax.axis_index("c")
    sid = lax.axis_index("s")
    wid = sid * NCORES + cid
    wbase = wid * WPW

    pltpu.sync_copy(lab.at[pl.ds(wbase, NFULL * CHUNK)], idx_all)
    pltpu.sync_copy(lab.at[pl.ds(wbase + NFULL * CHUNK, TAIL)], tidx_v)

    def fire_drain(round_base, nslots):
        gs = []
        for k in range(nslots):
            s = round_base + k * CHUNK
            gs.append(pltpu.async_copy(
                centers.at[idx_all.at[pl.ds(s, CHUNK)]], bufs[k], gsems[k]))
        ws = []
        for k in range(nslots):
            s = round_base + k * CHUNK
            gs[k].wait()
            ws.append(pltpu.async_copy(
                bufs[k], out.at[pl.ds(wbase + s, CHUNK)], wsems[k]))
        for k in range(nslots):
            ws[k].wait()

    def round_body(r, carry):
        fire_drain(r * NSLOT * CHUNK, NSLOT)
        return carry

    lax.fori_loop(0, NROUND, round_body, 0)

    left_base = NROUND * NSLOT * CHUNK
    gt = pltpu.async_copy(centers.at[tidx_v], tail_v, tsem)
    fire_drain(left_base, NLEFT)
    gt.wait()
    pltpu.sync_copy(tail_v, out.at[pl.ds(wbase + NFULL * CHUNK, TAIL)])


@functools.lru_cache(maxsize=None)
def _gather_kernel():
    mesh = plsc.VectorSubcoreMesh(core_axis_name="c", subcore_axis_name="s")
    return pl.kernel(
        _gather_body,
        out_type=jax.ShapeDtypeStruct((N, 128), jnp.float32),
        mesh=mesh,
        scratch_types=(
            [pltpu.VMEM((CHUNK, 128), jnp.float32)] * NSLOT
            + [
                pltpu.VMEM((TAIL, 128), jnp.float32),
                pltpu.VMEM((NFULL * CHUNK,), jnp.int32),
                pltpu.VMEM((TAIL,), jnp.int32),
            ]
            + [pltpu.SemaphoreType.DMA] * (NSLOT * 2 + 1)
        ),
    )


def _dist_body(feat_ref, gath_ref, lab_ref, cnt_ref, loss_ref, acc_ref):
    i = pl.program_id(0)

    @pl.when(i == 0)
    def _():
        acc_ref[...] = jnp.zeros((1, CP), jnp.float32)

    gp = lax.bitcast_convert_type(gath_ref[...], jnp.uint32)
    clo = lax.bitcast_convert_type(
        (gp & 0xFFFF).astype(jnp.uint16), jnp.bfloat16).astype(jnp.float32)
    chi = lax.bitcast_convert_type(
        (gp >> 16).astype(jnp.uint16), jnp.bfloat16).astype(jnp.float32)
    f = feat_ref[...]
    dlo = f[:, :128] - clo + 1e-6
    dhi = f[:, 128:] - chi + 1e-6
    dist = jnp.sqrt(jnp.sum(dlo * dlo, axis=1) +
                    jnp.sum(dhi * dhi, axis=1))
    lab = lab_ref[0, 0, :]
    onehot = (lab[:, None] ==
              lax.broadcasted_iota(jnp.int32, (BR, CP), 1)).astype(jnp.float32)
    acc_ref[...] += lax.dot_general(
        dist[None, :], onehot, (((1,), (0,)), ((), ())),
        preferred_element_type=jnp.float32)

    @pl.when(i == NB - 1)
    def _():
        cnt = cnt_ref[...][:, 0]
        ds = acc_ref[0, :]
        safe = jnp.where(cnt > 0, cnt, 1.0)
        loss_ref[...] = jnp.sum(jnp.where(cnt > 0, ds / safe, 0.0))[None, None]


def _distance_loss(features, gathered, lab3d, cnt):
    return pl.pallas_call(
        _dist_body,
        grid=(NB,),
        in_specs=[
            pl.BlockSpec((BR, D), lambda i: (i, 0)),
            pl.BlockSpec((BR, 128), lambda i: (i, 0)),
            pl.BlockSpec((1, 1, BR), lambda i: (i, 0, 0)),
            pl.BlockSpec((CP, 128), lambda i: (0, 0)),
        ],
        out_specs=pl.BlockSpec((1, 1), lambda i: (0, 0)),
        out_shape=jax.ShapeDtypeStruct((1, 1), jnp.float32),
        scratch_shapes=[pltpu.VMEM((1, CP), jnp.float32)],
    )(features, gathered, lab3d, cnt)


def kernel(features, labels):
    lab32 = labels.astype(jnp.int32)
    lab3d = lab32.reshape(NB, 1, BR)
    centers, cnt = _compute_centers(features, lab3d)
    gathered = _gather_kernel()(centers, lab32)
    loss = _distance_loss(features, gathered, lab3d, cnt)
    return loss[0, 0]


# speedup vs baseline: 2.2546x; 1.0980x over previous
import functools

import jax
import jax.numpy as jnp
from jax import lax
from jax.experimental import pallas as pl
from jax.experimental.pallas import tpu as pltpu
from jax.experimental.pallas import tpu_sc as plsc

N = 160000
D = 256
C = 1000
CP = 1024
CHUNK = 128
NCHUNKS = N // CHUNK
NCORES = 2
NSUB = 16
NW = NCORES * NSUB

BR = 2000
NB = N // BR


def _centers_body(feat_ref, lab_ref, cen_ref, cnt_ref, sums_acc, cnt_acc):
    i = pl.program_id(0)

    @pl.when(i == 0)
    def _():
        sums_acc[...] = jnp.zeros((CP, D), jnp.float32)
        cnt_acc[...] = jnp.zeros((CP, 128), jnp.float32)

    lab = lab_ref[0, 0, :]
    onehot = (lab[:, None] ==
              lax.broadcasted_iota(jnp.int32, (BR, CP), 1)).astype(jnp.float32)
    sums_acc[...] += lax.dot_general(
        onehot.astype(jnp.bfloat16), feat_ref[...].astype(jnp.bfloat16),
        (((0,), (0,)), ((), ())),
        preferred_element_type=jnp.float32)
    cnt_acc[...] += jnp.sum(onehot, axis=0)[:, None]

    @pl.when(i == NB - 1)
    def _():
        cnt = cnt_acc[...][:, 0:1]
        safe = jnp.where(cnt > 0, cnt, 1.0)
        cen = jnp.where(cnt > 0, sums_acc[...] / safe, 0.0)
        lo = lax.bitcast_convert_type(
            cen[:, :128].astype(jnp.bfloat16), jnp.uint16).astype(jnp.uint32)
        hi = lax.bitcast_convert_type(
            cen[:, 128:].astype(jnp.bfloat16), jnp.uint16).astype(jnp.uint32)
        cen_ref[...] = lax.bitcast_convert_type((hi << 16) | lo, jnp.float32)
        cnt_ref[...] = cnt_acc[...]


def _compute_centers(features, lab3d):
    return pl.pallas_call(
        _centers_body,
        grid=(NB,),
        in_specs=[
            pl.BlockSpec((BR, D), lambda i: (i, 0)),
            pl.BlockSpec((1, 1, BR), lambda i: (i, 0, 0)),
        ],
        out_specs=[
            pl.BlockSpec((CP, 128), lambda i: (0, 0)),
            pl.BlockSpec((CP, 128), lambda i: (0, 0)),
        ],
        out_shape=[
            jax.ShapeDtypeStruct((CP, 128), jnp.float32),
            jax.ShapeDtypeStruct((CP, 128), jnp.float32),
        ],
        scratch_shapes=[
            pltpu.VMEM((CP, D), jnp.float32),
            pltpu.VMEM((CP, 128), jnp.float32),
        ],
    )(features, lab3d)


NSLICE = 5
SROWS = N // NSLICE
WPW = SROWS // NW
NFULL = WPW // CHUNK
TAIL = WPW - NFULL * CHUNK
NSLOT = 4
NROUND = NFULL // NSLOT
NLEFT = NFULL - NROUND * NSLOT


def _gather_body(sbase, centers, lab, out, *refs):
    bufs = refs[0:NSLOT]
    tail_v = refs[NSLOT]
    idx_all = refs[NSLOT + 1]
    tidx_v = refs[NSLOT + 2]
    gsems = refs[NSLOT + 3:NSLOT + 3 + NSLOT]
    wsems = refs[NSLOT * 2 + 3:NSLOT * 2 + 3 + NSLOT]
    tsem = refs[NSLOT * 3 + 3]

    cid = lax.axis_index("c")
    sid = lax.axis_index("s")
    wid = sid * NCORES + cid
    obase = wid * WPW
    wbase = sbase + obase

    pltpu.sync_copy(lab.at[pl.ds(wbase, NFULL * CHUNK)], idx_all)
    pltpu.sync_copy(lab.at[pl.ds(wbase + NFULL * CHUNK, TAIL)], tidx_v)

    def fire_drain(round_base, nslots):
        gs = []
        for k in range(nslots):
            s = round_base + k * CHUNK
            gs.append(pltpu.async_copy(
                centers.at[idx_all.at[pl.ds(s, CHUNK)]], bufs[k], gsems[k]))
        ws = []
        for k in range(nslots):
            s = round_base + k * CHUNK
            gs[k].wait()
            ws.append(pltpu.async_copy(
                bufs[k], out.at[pl.ds(obase + s, CHUNK)], wsems[k]))
        for k in range(nslots):
            ws[k].wait()

    def round_body(r, carry):
        fire_drain(r * NSLOT * CHUNK, NSLOT)
        return carry

    lax.fori_loop(0, NROUND, round_body, 0)

    left_base = NROUND * NSLOT * CHUNK
    gt = pltpu.async_copy(centers.at[tidx_v], tail_v, tsem)
    fire_drain(left_base, NLEFT)
    gt.wait()
    pltpu.sync_copy(tail_v, out.at[pl.ds(obase + NFULL * CHUNK, TAIL)])


@functools.lru_cache(maxsize=None)
def _gather_kernel(slice_idx):
    mesh = plsc.VectorSubcoreMesh(core_axis_name="c", subcore_axis_name="s")
    return pl.kernel(
        functools.partial(_gather_body, slice_idx * SROWS),
        out_type=jax.ShapeDtypeStruct((SROWS, 128), jnp.float32),
        mesh=mesh,
        scratch_types=(
            [pltpu.VMEM((CHUNK, 128), jnp.float32)] * NSLOT
            + [
                pltpu.VMEM((TAIL, 128), jnp.float32),
                pltpu.VMEM((NFULL * CHUNK,), jnp.int32),
                pltpu.VMEM((TAIL,), jnp.int32),
            ]
            + [pltpu.SemaphoreType.DMA] * (NSLOT * 2 + 1)
        ),
    )


NBS = SROWS // BR


def _dist_body(feat_ref, gath_ref, lab_ref, part_ref):
    i = pl.program_id(0)

    @pl.when(i == 0)
    def _():
        part_ref[...] = jnp.zeros((1, CP), jnp.float32)

    gp = lax.bitcast_convert_type(gath_ref[...], jnp.uint32)
    clo = lax.bitcast_convert_type(
        (gp & 0xFFFF).astype(jnp.uint16), jnp.bfloat16).astype(jnp.float32)
    chi = lax.bitcast_convert_type(
        (gp >> 16).astype(jnp.uint16), jnp.bfloat16).astype(jnp.float32)
    f = feat_ref[...]
    dlo = f[:, :128] - clo + 1e-6
    dhi = f[:, 128:] - chi + 1e-6
    dist = jnp.sqrt(jnp.sum(dlo * dlo, axis=1) +
                    jnp.sum(dhi * dhi, axis=1))
    lab = lab_ref[0, 0, :]
    onehot = (lab[:, None] ==
              lax.broadcasted_iota(jnp.int32, (BR, CP), 1)).astype(jnp.float32)
    part_ref[...] += lax.dot_general(
        dist[None, :], onehot, (((1,), (0,)), ((), ())),
        preferred_element_type=jnp.float32)


def _dist_partial(s, features, gathered, lab3d):
    return pl.pallas_call(
        _dist_body,
        grid=(NBS,),
        in_specs=[
            pl.BlockSpec((BR, D), lambda i, s=s: (s * NBS + i, 0)),
            pl.BlockSpec((BR, 128), lambda i: (i, 0)),
            pl.BlockSpec((1, 1, BR), lambda i, s=s: (s * NBS + i, 0, 0)),
        ],
        out_specs=pl.BlockSpec((1, CP), lambda i: (0, 0)),
        out_shape=jax.ShapeDtypeStruct((1, CP), jnp.float32),
    )(features, gathered, lab3d)


def _final_body(parts_ref, cnt_ref, loss_ref):
    cnt = cnt_ref[...][:, 0]
    ds = jnp.sum(parts_ref[...], axis=0)
    safe = jnp.where(cnt > 0, cnt, 1.0)
    loss_ref[...] = jnp.sum(jnp.where(cnt > 0, ds / safe, 0.0))[None, None]


def _final_loss(parts, cnt):
    return pl.pallas_call(
        _final_body,
        out_shape=jax.ShapeDtypeStruct((1, 1), jnp.float32),
    )(parts, cnt)


def kernel(features, labels):
    lab32 = labels.astype(jnp.int32)
    lab3d = lab32.reshape(NB, 1, BR)
    centers, cnt = _compute_centers(features, lab3d)
    parts = []
    for s in range(NSLICE):
        gathered = _gather_kernel(s)(centers, lab32)
        parts.append(_dist_partial(s, features, gathered, lab3d))
    loss = _final_loss(jnp.concatenate(parts, axis=0), cnt)
    return loss[0, 0]


# speedup vs baseline: 2.2613x; 1.0030x over previous
import functools

import jax
import jax.numpy as jnp
from jax import lax
from jax.experimental import pallas as pl
from jax.experimental.pallas import tpu as pltpu
from jax.experimental.pallas import tpu_sc as plsc

N = 160000
D = 256
C = 1000
CP = 1024
CHUNK = 128
NCHUNKS = N // CHUNK
NCORES = 2
NSUB = 16
NW = NCORES * NSUB

BR = 2000
NB = N // BR


def _centers_body(feat_ref, lab_ref, cen_ref, cnt_ref, sums_acc, cnt_acc):
    i = pl.program_id(0)

    @pl.when(i == 0)
    def _():
        sums_acc[...] = jnp.zeros((CP, D), jnp.float32)
        cnt_acc[...] = jnp.zeros((CP, 128), jnp.float32)

    lab = lab_ref[0, 0, :]
    fbf = feat_ref[...].astype(jnp.bfloat16)
    b0 = jnp.minimum(jnp.min(lab) // 8 * 8, CP - 256)

    def window(k):
        base = pl.multiple_of(b0 + 256 * k, 8)
        rel = lab - base

        @pl.when(jnp.any((rel >= 0) & (rel < 256)))
        def _():
            oh = (rel[:, None] ==
                  lax.broadcasted_iota(jnp.int32, (BR, 256), 1)
                  ).astype(jnp.float32)
            part = lax.dot_general(
                oh.astype(jnp.bfloat16), fbf, (((0,), (0,)), ((), ())),
                preferred_element_type=jnp.float32)
            sums_acc[pl.ds(base, 256), :] += part
            cnt_acc[pl.ds(base, 256), :] += jnp.sum(oh, axis=0)[:, None]

    for k in range(4):
        window(k)

    @pl.when(i == NB - 1)
    def _():
        cnt = cnt_acc[...][:, 0:1]
        safe = jnp.where(cnt > 0, cnt, 1.0)
        cen = jnp.where(cnt > 0, sums_acc[...] / safe, 0.0)
        lo = lax.bitcast_convert_type(
            cen[:, :128].astype(jnp.bfloat16), jnp.uint16).astype(jnp.uint32)
        hi = lax.bitcast_convert_type(
            cen[:, 128:].astype(jnp.bfloat16), jnp.uint16).astype(jnp.uint32)
        cen_ref[...] = lax.bitcast_convert_type((hi << 16) | lo, jnp.float32)
        cnt_ref[...] = cnt_acc[...]


def _compute_centers(features, lab3d):
    return pl.pallas_call(
        _centers_body,
        grid=(NB,),
        in_specs=[
            pl.BlockSpec((BR, D), lambda i: (i, 0)),
            pl.BlockSpec((1, 1, BR), lambda i: (i, 0, 0)),
        ],
        out_specs=[
            pl.BlockSpec((CP, 128), lambda i: (0, 0)),
            pl.BlockSpec((CP, 128), lambda i: (0, 0)),
        ],
        out_shape=[
            jax.ShapeDtypeStruct((CP, 128), jnp.float32),
            jax.ShapeDtypeStruct((CP, 128), jnp.float32),
        ],
        scratch_shapes=[
            pltpu.VMEM((CP, D), jnp.float32),
            pltpu.VMEM((CP, 128), jnp.float32),
        ],
    )(features, lab3d)


NSLICE = 5
SROWS = N // NSLICE
WPW = SROWS // NW
NFULL = WPW // CHUNK
TAIL = WPW - NFULL * CHUNK
NSLOT = 4
NROUND = NFULL // NSLOT
NLEFT = NFULL - NROUND * NSLOT


def _gather_body(sbase, centers, lab, out, *refs):
    bufs = refs[0:NSLOT]
    tail_v = refs[NSLOT]
    idx_all = refs[NSLOT + 1]
    tidx_v = refs[NSLOT + 2]
    gsems = refs[NSLOT + 3:NSLOT + 3 + NSLOT]
    wsems = refs[NSLOT * 2 + 3:NSLOT * 2 + 3 + NSLOT]
    tsem = refs[NSLOT * 3 + 3]

    cid = lax.axis_index("c")
    sid = lax.axis_index("s")
    wid = sid * NCORES + cid
    obase = wid * WPW
    wbase = sbase + obase

    pltpu.sync_copy(lab.at[pl.ds(wbase, NFULL * CHUNK)], idx_all)
    pltpu.sync_copy(lab.at[pl.ds(wbase + NFULL * CHUNK, TAIL)], tidx_v)

    def fire_drain(round_base, nslots):
        gs = []
        for k in range(nslots):
            s = round_base + k * CHUNK
            gs.append(pltpu.async_copy(
                centers.at[idx_all.at[pl.ds(s, CHUNK)]], bufs[k], gsems[k]))
        ws = []
        for k in range(nslots):
            s = round_base + k * CHUNK
            gs[k].wait()
            ws.append(pltpu.async_copy(
                bufs[k], out.at[pl.ds(obase + s, CHUNK)], wsems[k]))
        for k in range(nslots):
            ws[k].wait()

    def round_body(r, carry):
        fire_drain(r * NSLOT * CHUNK, NSLOT)
        return carry

    lax.fori_loop(0, NROUND, round_body, 0)

    left_base = NROUND * NSLOT * CHUNK
    gt = pltpu.async_copy(centers.at[tidx_v], tail_v, tsem)
    fire_drain(left_base, NLEFT)
    gt.wait()
    pltpu.sync_copy(tail_v, out.at[pl.ds(obase + NFULL * CHUNK, TAIL)])


@functools.lru_cache(maxsize=None)
def _gather_kernel(slice_idx):
    mesh = plsc.VectorSubcoreMesh(core_axis_name="c", subcore_axis_name="s")
    return pl.kernel(
        functools.partial(_gather_body, slice_idx * SROWS),
        out_type=jax.ShapeDtypeStruct((SROWS, 128), jnp.float32),
        mesh=mesh,
        scratch_types=(
            [pltpu.VMEM((CHUNK, 128), jnp.float32)] * NSLOT
            + [
                pltpu.VMEM((TAIL, 128), jnp.float32),
                pltpu.VMEM((NFULL * CHUNK,), jnp.int32),
                pltpu.VMEM((TAIL,), jnp.int32),
            ]
            + [pltpu.SemaphoreType.DMA] * (NSLOT * 2 + 1)
        ),
    )


NBS = SROWS // BR


def _dist_body(feat_ref, gath_ref, lab_ref, part_ref):
    i = pl.program_id(0)

    @pl.when(i == 0)
    def _():
        part_ref[...] = jnp.zeros((1, CP), jnp.float32)

    gp = lax.bitcast_convert_type(gath_ref[...], jnp.uint32)
    clo = lax.bitcast_convert_type(
        (gp & 0xFFFF).astype(jnp.uint16), jnp.bfloat16).astype(jnp.float32)
    chi = lax.bitcast_convert_type(
        (gp >> 16).astype(jnp.uint16), jnp.bfloat16).astype(jnp.float32)
    f = feat_ref[...]
    dlo = f[:, :128] - clo + 1e-6
    dhi = f[:, 128:] - chi + 1e-6
    dist = jnp.sqrt(jnp.sum(dlo * dlo, axis=1) +
                    jnp.sum(dhi * dhi, axis=1))
    lab = lab_ref[0, 0, :]
    onehot = (lab[:, None] ==
              lax.broadcasted_iota(jnp.int32, (BR, CP), 1)).astype(jnp.float32)
    part_ref[...] += lax.dot_general(
        dist[None, :], onehot, (((1,), (0,)), ((), ())),
        preferred_element_type=jnp.float32)


def _dist_partial(s, features, gathered, lab3d):
    return pl.pallas_call(
        _dist_body,
        grid=(NBS,),
        in_specs=[
            pl.BlockSpec((BR, D), lambda i, s=s: (s * NBS + i, 0)),
            pl.BlockSpec((BR, 128), lambda i: (i, 0)),
            pl.BlockSpec((1, 1, BR), lambda i, s=s: (s * NBS + i, 0, 0)),
        ],
        out_specs=pl.BlockSpec((1, CP), lambda i: (0, 0)),
        out_shape=jax.ShapeDtypeStruct((1, CP), jnp.float32),
    )(features, gathered, lab3d)


def _final_body(parts_ref, cnt_ref, loss_ref):
    cnt = cnt_ref[...][:, 0]
    ds = jnp.sum(parts_ref[...], axis=0)
    safe = jnp.where(cnt > 0, cnt, 1.0)
    loss_ref[...] = jnp.sum(jnp.where(cnt > 0, ds / safe, 0.0))[None, None]


def _final_loss(parts, cnt):
    return pl.pallas_call(
        _final_body,
        out_shape=jax.ShapeDtypeStruct((1, 1), jnp.float32),
    )(parts, cnt)


def kernel(features, labels):
    lab32 = labels.astype(jnp.int32)
    lab3d = lab32.reshape(NB, 1, BR)
    centers, cnt = _compute_centers(features, lab3d)
    parts = []
    for s in range(NSLICE):
        gathered = _gather_kernel(s)(centers, lab32)
        parts.append(_dist_partial(s, features, gathered, lab3d))
    loss = _final_loss(jnp.concatenate(parts, axis=0), cnt)
    return loss[0, 0]
